# baseline (speedup 1.0000x reference)
import functools

import jax
import jax.numpy as jnp
from jax import lax
from jax.experimental import pallas as pl
from jax.experimental.pallas import tpu as pltpu
from jax.experimental.pallas import tpu_sc as plsc

B = 16384
N_EMB = 20
NC = 2
NS = 16
NW = NC * NS
BPW = B // NW
CHUNK = 128
NCHUNK = BPW // CHUNK
L = 16
GPC = CHUNK // L
NROWS = 1000000 * N_EMB // 128


def _sc_body(tab128, bias_ref, idx_hbm, c_hbm, wb_hbm, out_hbm,
             idx_v, qa_v, qb_v, c_v, bias_v, out_v, wb_v, rowsA, rowsB, sem):
    wid = lax.axis_index("s") * NC + lax.axis_index("c")

    pltpu.sync_copy(idx_hbm.at[wid], idx_v)
    pltpu.sync_copy(c_hbm.at[wid], c_v)
    pltpu.sync_copy(wb_hbm, wb_v)

    wbv = [wb_v[pl.ds(16 * j, 16)] for j in range(3)]
    wbs = [wbv[j // 16][j % 16] for j in range(2 * N_EMB)]
    w0s = wbs[:N_EMB]
    b0s = wbs[N_EMB:]
    lanes = lax.iota(jnp.int32, L)

    for k in range(NCHUNK):
        def mkq(t, _, k=k):
            r = idx_v[k, pl.ds(t * L, L)]
            f = r * N_EMB
            q = f // 128
            qa_v[pl.ds(t * L, L)] = q
            qb_v[pl.ds(t * L, L)] = jnp.minimum(q + 1, NROWS - 1)
            return _
        lax.fori_loop(0, GPC, mkq, 0)

        ca = pltpu.async_copy(tab128.at[qa_v], rowsA, sem)
        cb = pltpu.async_copy(tab128.at[qb_v], rowsB, sem)
        cbias = pltpu.async_copy(
            bias_ref.at[idx_v.at[k]], bias_v.at[pl.ds(k * CHUNK, CHUNK)], sem)
        ca.wait()
        cb.wait()
        cbias.wait()

        def group(t, _, k=k):
            tok = t * L + lanes
            r = idx_v[k, pl.ds(t * L, L)]
            q = qa_v[pl.ds(t * L, L)]
            off = r * N_EMB - q * 128
            c = c_v[pl.ds(k * CHUNK + t * L, L)]
            acc = bias_v[pl.ds(k * CHUNK + t * L, L)]
            zero = jnp.zeros((L,), jnp.int32)
            for e in range(N_EMB):
                oe = off + e
                in_a = oe < jnp.full((L,), 128, jnp.int32)
                w_a = plsc.load_gather(
                    rowsA, [tok, jnp.where(in_a, oe, zero)], mask=in_a)
                w_b = plsc.load_gather(
                    rowsB, [tok, jnp.where(in_a, zero, oe - 128)],
                    mask=~in_a)
                w_e = jnp.where(in_a, w_a, w_b)
                h_e = jnp.maximum(c * w0s[e] + b0s[e], 0.0)
                acc = acc + h_e * w_e
            out_v[pl.ds(k * CHUNK + t * L, L)] = acc
            return _
        lax.fori_loop(0, GPC, group, 0)

    pltpu.sync_copy(out_v, out_hbm.at[wid])


@jax.jit
def _run(table, bias, idx, coords, wb):
    mesh = plsc.VectorSubcoreMesh(core_axis_name="c", subcore_axis_name="s")
    f = functools.partial(
        pl.kernel,
        mesh=mesh,
        out_type=jax.ShapeDtypeStruct((NW, BPW), jnp.float32),
        scratch_types=[
            pltpu.VMEM((NCHUNK, CHUNK), jnp.int32),
            pltpu.VMEM((CHUNK,), jnp.int32),
            pltpu.VMEM((CHUNK,), jnp.int32),
            pltpu.VMEM((BPW,), jnp.float32),
            pltpu.VMEM((BPW,), jnp.float32),
            pltpu.VMEM((BPW,), jnp.float32),
            pltpu.VMEM((48,), jnp.float32),
            pltpu.VMEM((CHUNK, 128), jnp.float32),
            pltpu.VMEM((CHUNK, 128), jnp.float32),
            pltpu.SemaphoreType.DMA,
        ],
        compiler_params=pltpu.CompilerParams(
            needs_layout_passes=False, use_tc_tiling_on_sc=False),
    )(_sc_body)
    return f(table, bias, idx, coords, wb)


def kernel(coordinates, region_ix, W0, b0, weight1_table, bias1_table):
    table = weight1_table.reshape(NROWS, 128)
    bias = bias1_table.reshape(-1)
    idx = region_ix.astype(jnp.int32).reshape(NW, NCHUNK, CHUNK)
    coords = coordinates.reshape(NW, BPW)
    wb = jnp.concatenate(
        [W0.reshape(-1) / 20000.0, b0, jnp.zeros((8,), jnp.float32)])
    out = _run(table, bias, idx, coords, wb)
    return out.reshape(B, 1)


# speedup vs baseline: 2.2700x; 2.2700x over previous
import functools

import jax
import jax.numpy as jnp
from jax import lax
from jax.experimental import pallas as pl
from jax.experimental.pallas import tpu as pltpu
from jax.experimental.pallas import tpu_sc as plsc

B = 16384
N_REG = 1000000
N_EMB = 20
NC = 2
NS = 16
NW = NC * NS
BPW = B // NW
CHUNK = 128
NCHUNK = BPW // CHUNK
L = 16


def _sc_body(*refs):
    tab_es = refs[:N_EMB]
    (bias_ref, idx_hbm, c_hbm, wb_hbm, out_hbm,
     idx_v, cols_v, c_v, bias_v, out_v, wb_v, sem) = refs[N_EMB:]
    wid = lax.axis_index("s") * NC + lax.axis_index("c")

    pltpu.sync_copy(idx_hbm.at[wid], idx_v)
    pltpu.sync_copy(c_hbm.at[wid], c_v)
    pltpu.sync_copy(wb_hbm, wb_v)

    copies = []
    for k in range(NCHUNK):
        for e in range(N_EMB):
            copies.append(pltpu.async_copy(
                tab_es[e].at[idx_v.at[k]],
                cols_v.at[e, pl.ds(k * CHUNK, CHUNK)], sem))
        copies.append(pltpu.async_copy(
            bias_ref.at[idx_v.at[k]], bias_v.at[pl.ds(k * CHUNK, CHUNK)],
            sem))
    for c in copies:
        c.wait()

    wbv = [wb_v[pl.ds(16 * j, 16)] for j in range(3)]
    wbs = [wbv[j // 16][j % 16] for j in range(2 * N_EMB)]
    w0s = wbs[:N_EMB]
    b0s = wbs[N_EMB:]

    def group(t, _):
        c = c_v[pl.ds(t * L, L)]
        acc = bias_v[pl.ds(t * L, L)]
        for e in range(N_EMB):
            w_e = cols_v[e, pl.ds(t * L, L)]
            h_e = jnp.maximum(c * w0s[e] + b0s[e], 0.0)
            acc = acc + h_e * w_e
        out_v[pl.ds(t * L, L)] = acc
        return _
    lax.fori_loop(0, BPW // L, group, 0)

    pltpu.sync_copy(out_v, out_hbm.at[wid])


@jax.jit
def _run(tab_es, bias, idx, coords, wb):
    mesh = plsc.VectorSubcoreMesh(core_axis_name="c", subcore_axis_name="s")
    f = functools.partial(
        pl.kernel,
        mesh=mesh,
        out_type=jax.ShapeDtypeStruct((NW, BPW), jnp.float32),
        scratch_types=[
            pltpu.VMEM((NCHUNK, CHUNK), jnp.int32),
            pltpu.VMEM((N_EMB, BPW), jnp.float32),
            pltpu.VMEM((BPW,), jnp.float32),
            pltpu.VMEM((BPW,), jnp.float32),
            pltpu.VMEM((BPW,), jnp.float32),
            pltpu.VMEM((48,), jnp.float32),
            pltpu.SemaphoreType.DMA,
        ],
        compiler_params=pltpu.CompilerParams(
            needs_layout_passes=False, use_tc_tiling_on_sc=False),
    )(_sc_body)
    return f(*tab_es, bias, idx, coords, wb)


def kernel(coordinates, region_ix, W0, b0, weight1_table, bias1_table):
    tab_es = tuple(weight1_table[:, e, 0] for e in range(N_EMB))
    bias = bias1_table.reshape(-1)
    idx = region_ix.astype(jnp.int32).reshape(NW, NCHUNK, CHUNK)
    coords = coordinates.reshape(NW, BPW)
    wb = jnp.concatenate(
        [W0.reshape(-1) / 20000.0, b0, jnp.zeros((8,), jnp.float32)])
    out = _run(tab_es, bias, idx, coords, wb)
    return out.reshape(B, 1)
